# Initial kernel scaffold; baseline (speedup 1.0000x reference)
import jax
import jax.numpy as jnp
from jax.experimental import pallas as pl


def kernel(edge_index, h, e, snorm_n, snorm_e, W_embed, W_layers, b_layers, W_ro, W_pred, b_pred):
    raise NotImplementedError("write your pallas kernel here")


# speedup vs baseline: 4.6213x; 4.6213x over previous
import functools

import jax
import jax.numpy as jnp
from jax import lax
from jax.experimental import pallas as pl
from jax.experimental.pallas import tpu as pltpu
from jax.experimental.pallas import tpu_sc as plsc

_NC = 2
_NS = 16
_NW = _NC * _NS
_CHUNK = 128


def _make_sc_agg(n_pad, d, n_chunks, gather):
    rows_per_tile = n_pad // _NS
    mesh = plsc.VectorSubcoreMesh(core_axis_name="c", subcore_axis_name="s")

    @functools.partial(
        pl.kernel,
        mesh=mesh,
        out_type=(jax.ShapeDtypeStruct((n_pad, d), jnp.float32),
                  jax.ShapeDtypeStruct((n_pad, d), jnp.float32)),
        scratch_types=[
            pltpu.VMEM((n_chunks, _CHUNK), jnp.int32),
            pltpu.VMEM((n_chunks, _CHUNK), jnp.int32),
            pltpu.VMEM((_CHUNK, d), jnp.float32),
            pltpu.VMEM_SHARED((n_pad, d), jnp.float32),
            pltpu.SemaphoreType.DMA,
        ],
    )
    def agg_kernel(table_hbm, src_hbm, dst_hbm, zeros_hbm, out0_hbm,
                   out1_hbm, src_v, dst_v, rows_v, acc_sh, sem):
        cid = lax.axis_index("c")
        sid = lax.axis_index("s")
        wid = cid * _NS + sid
        r0 = sid * rows_per_tile
        pltpu.sync_copy(zeros_hbm.at[pl.ds(r0, rows_per_tile)],
                        acc_sh.at[pl.ds(r0, rows_per_tile)])
        pltpu.sync_copy(src_hbm.at[wid], src_v)
        pltpu.sync_copy(dst_hbm.at[wid], dst_v)
        if not gather:
            pltpu.sync_copy(table_hbm, rows_v)
        plsc.subcore_barrier()

        def body(j, carry):
            if gather:
                pltpu.async_copy(table_hbm.at[src_v.at[j]], rows_v, sem).wait()
            pltpu.sync_copy(rows_v, acc_sh.at[dst_v.at[j]], add=True)
            return carry

        lax.fori_loop(0, n_chunks, body, 0)
        plsc.subcore_barrier()
        @pl.when(cid == 0)
        def _():
            pltpu.sync_copy(acc_sh.at[pl.ds(r0, rows_per_tile)],
                            out0_hbm.at[pl.ds(r0, rows_per_tile)])

        @pl.when(cid == 1)
        def _():
            pltpu.sync_copy(acc_sh.at[pl.ds(r0, rows_per_tile)],
                            out1_hbm.at[pl.ds(r0, rows_per_tile)])

    return agg_kernel


def _embed_body(h_ref, w_ref, d0_ref, d1_ref, o_ref, dinv_ref):
    o_ref[...] = jnp.dot(h_ref[...], w_ref[...],
                         preferred_element_type=jnp.float32)
    deg = d0_ref[:, :1] + d1_ref[:, :1]
    dinv_ref[...] = 1.0 / jnp.maximum(deg, 1.0)


def _layer_body(h_ref, p0_ref, p1_ref, dinv_ref, sn_ref, ws_ref, wn_ref,
                b_ref, o_ref):
    agg = (p0_ref[...] + p1_ref[...]) * dinv_ref[...]
    z = (jnp.dot(h_ref[...], ws_ref[...], preferred_element_type=jnp.float32)
         + jnp.dot(agg, wn_ref[...], preferred_element_type=jnp.float32)
         + b_ref[...])
    o_ref[...] = jnp.maximum(z, 0.0) * sn_ref[...] + h_ref[...]


def _make_readout_body(inv_n):
    def _readout_body(h_ref, wro_ref, wp_ref, bp_ref, o_ref):
        s = jnp.sum(h_ref[...], axis=0, keepdims=True) * inv_n
        v = jnp.dot(s, wro_ref[...], preferred_element_type=jnp.float32)
        o_ref[...] = jnp.dot(v, wp_ref[...],
                             preferred_element_type=jnp.float32) + bp_ref[...]
    return _readout_body


def kernel(edge_index, h, e, snorm_n, snorm_e, W_embed, W_layers, b_layers,
           W_ro, W_pred, b_pred):
    n, d_in = h.shape
    hd = W_embed.shape[1]
    n_layers = W_layers.shape[0]
    n_edges = edge_index.shape[1]

    bn = 400 if n % 400 == 0 else n
    n_pad = -(-(n + 1) // 128) * 128
    n_chunks = -(-n_edges // (_CHUNK * _NW))
    e_pad = n_chunks * _CHUNK * _NW

    src = jnp.pad(edge_index[0], (0, e_pad - n_edges))
    dst = jnp.pad(edge_index[1], (0, e_pad - n_edges), constant_values=n)
    src3 = src.reshape(_NW, n_chunks, _CHUNK)
    dst3 = dst.reshape(_NW, n_chunks, _CHUNK)

    zeros_h = jnp.zeros((n_pad, hd), jnp.float32)
    ones_t = jnp.ones((_CHUNK, hd), jnp.float32)

    agg_fn = _make_sc_agg(n_pad, hd, n_chunks, gather=True)
    deg_fn = _make_sc_agg(n_pad, hd, n_chunks, gather=False)

    nb = n // bn

    deg0, deg1 = deg_fn(ones_t, src3, dst3, zeros_h)

    h0, dinv = pl.pallas_call(
        _embed_body,
        grid=(nb,),
        in_specs=[
            pl.BlockSpec((bn, d_in), lambda i: (i, 0)),
            pl.BlockSpec((d_in, hd), lambda i: (0, 0)),
            pl.BlockSpec((bn, hd), lambda i: (i, 0)),
            pl.BlockSpec((bn, hd), lambda i: (i, 0)),
        ],
        out_specs=[
            pl.BlockSpec((bn, hd), lambda i: (i, 0)),
            pl.BlockSpec((bn, 1), lambda i: (i, 0)),
        ],
        out_shape=[
            jax.ShapeDtypeStruct((n, hd), jnp.float32),
            jax.ShapeDtypeStruct((n, 1), jnp.float32),
        ],
    )(h, W_embed, deg0, deg1)

    hcur = h0
    for l in range(n_layers):
        p0, p1 = agg_fn(hcur, src3, dst3, zeros_h)
        hcur = pl.pallas_call(
            _layer_body,
            grid=(nb,),
            in_specs=[
                pl.BlockSpec((bn, hd), lambda i: (i, 0)),
                pl.BlockSpec((bn, hd), lambda i: (i, 0)),
                pl.BlockSpec((bn, hd), lambda i: (i, 0)),
                pl.BlockSpec((bn, 1), lambda i: (i, 0)),
                pl.BlockSpec((bn, 1), lambda i: (i, 0)),
                pl.BlockSpec((hd, hd), lambda i: (0, 0)),
                pl.BlockSpec((hd, hd), lambda i: (0, 0)),
                pl.BlockSpec((1, hd), lambda i: (0, 0)),
            ],
            out_specs=pl.BlockSpec((bn, hd), lambda i: (i, 0)),
            out_shape=jax.ShapeDtypeStruct((n, hd), jnp.float32),
        )(hcur, p0, p1, dinv, snorm_n,
          W_layers[l, :hd], W_layers[l, hd:], b_layers[l][None, :])

    out = pl.pallas_call(
        _make_readout_body(1.0 / n),
        out_shape=jax.ShapeDtypeStruct((1, 1), jnp.float32),
    )(hcur, W_ro, W_pred, b_pred[None, :])
    return out


def _gcd(a, b):
    while b:
        a, b = b, a % b
    return a
